# Initial kernel scaffold; baseline (speedup 1.0000x reference)
import jax
import jax.numpy as jnp
from jax.experimental import pallas as pl


def kernel(first, second, table, W, b):
    raise NotImplementedError("write your pallas kernel here")


# speedup vs baseline: 4.7750x; 4.7750x over previous
import functools

import jax
import jax.numpy as jnp
from jax import lax
from jax.experimental import pallas as pl
from jax.experimental.pallas import tpu as pltpu
from jax.experimental.pallas import tpu_sc as plsc

_VOCAB = 800
_EMB = 128
_BATCH = 16384

_NC = 2
_NS = 16
_NW = _NC * _NS
_L = 16
_BPW = _BATCH // _NW


def _tc_scores_body(table_ref, w_ref, b_ref, t1_ref, t2_ref):
    tab = table_ref[...]
    w1 = w_ref[0:1, :]
    w2 = w_ref[1:2, :]
    t1_ref[...] = jnp.sum(tab * w1, axis=1) + b_ref[0]
    t2_ref[...] = jnp.sum(tab * w2, axis=1)


def _tc_scores(table, w2row, b):
    return pl.pallas_call(
        _tc_scores_body,
        out_shape=[
            jax.ShapeDtypeStruct((_VOCAB,), jnp.float32),
            jax.ShapeDtypeStruct((_VOCAB,), jnp.float32),
        ],
        in_specs=[
            pl.BlockSpec(memory_space=pltpu.VMEM),
            pl.BlockSpec(memory_space=pltpu.VMEM),
            pl.BlockSpec(memory_space=pltpu.SMEM),
        ],
    )(table, w2row, b)


def _sc_gather_body(t1_hbm, t2_hbm, first_hbm, second_hbm, out_hbm,
                    t1_v, t2_v, f_v, s_v, o_v):
    wid = lax.axis_index("s") * _NC + lax.axis_index("c")
    base = wid * _BPW
    pltpu.sync_copy(t1_hbm, t1_v)
    pltpu.sync_copy(t2_hbm, t2_v)
    pltpu.sync_copy(first_hbm.at[pl.ds(base, _BPW)], f_v)
    pltpu.sync_copy(second_hbm.at[pl.ds(base, _BPW)], s_v)

    def body(i, carry):
        off = i * _L
        idx1 = f_v[pl.ds(off, _L)]
        idx2 = s_v[pl.ds(off, _L)]
        a = plsc.load_gather(t1_v, [idx1])
        c = plsc.load_gather(t2_v, [idx2])
        x = a + c
        o_v[pl.ds(off, _L)] = 1.0 / (1.0 + jnp.exp(-x))
        return carry

    lax.fori_loop(0, _BPW // _L, body, 0)
    pltpu.sync_copy(o_v, out_hbm.at[pl.ds(base, _BPW)])


_sc_gather = functools.partial(
    pl.kernel,
    out_type=jax.ShapeDtypeStruct((_BATCH,), jnp.float32),
    mesh=plsc.VectorSubcoreMesh(core_axis_name="c", subcore_axis_name="s"),
    compiler_params=pltpu.CompilerParams(needs_layout_passes=False),
    scratch_types=[
        pltpu.VMEM((_VOCAB,), jnp.float32),
        pltpu.VMEM((_VOCAB,), jnp.float32),
        pltpu.VMEM((_BPW,), jnp.int32),
        pltpu.VMEM((_BPW,), jnp.int32),
        pltpu.VMEM((_BPW,), jnp.float32),
    ],
)(_sc_gather_body)


@jax.jit
def kernel(first, second, table, W, b):
    w2row = W.reshape(2, _EMB)
    t1, t2 = _tc_scores(table, w2row, b)
    out = _sc_gather(t1, t2, first.astype(jnp.int32), second.astype(jnp.int32))
    return out.reshape(_BATCH, 1)


# speedup vs baseline: 5.0753x; 1.0629x over previous
import functools

import jax
import jax.numpy as jnp
from jax import lax
from jax.experimental import pallas as pl
from jax.experimental.pallas import tpu as pltpu
from jax.experimental.pallas import tpu_sc as plsc

_VOCAB = 800
_EMB = 128
_BATCH = 16384

_NC = 2
_NS = 16
_NW = _NC * _NS
_L = 16
_BPW = _BATCH // _NW


def _tc_scores_body(table_ref, w_ref, b_ref, t1_ref, t2_ref):
    tab = table_ref[...]
    w1 = w_ref[0:1, :]
    w2 = w_ref[1:2, :]
    t1_ref[...] = jnp.sum(tab * w1, axis=1) + b_ref[0]
    t2_ref[...] = jnp.sum(tab * w2, axis=1)


def _tc_scores(table, w2row, b):
    return pl.pallas_call(
        _tc_scores_body,
        out_shape=[
            jax.ShapeDtypeStruct((_VOCAB,), jnp.float32),
            jax.ShapeDtypeStruct((_VOCAB,), jnp.float32),
        ],
        in_specs=[
            pl.BlockSpec(memory_space=pltpu.VMEM),
            pl.BlockSpec(memory_space=pltpu.VMEM),
            pl.BlockSpec(memory_space=pltpu.SMEM),
        ],
    )(table, w2row, b)


def _sc_gather_body(t1_hbm, t2_hbm, first_hbm, second_hbm, out_hbm,
                    t1_v, t2_v, f_v, s_v, o_v, sem):
    wid = lax.axis_index("s") * _NC + lax.axis_index("c")
    base = wid * _BPW
    c1 = pltpu.async_copy(t1_hbm, t1_v, sem)
    c2 = pltpu.async_copy(t2_hbm, t2_v, sem)
    c3 = pltpu.async_copy(first_hbm.at[pl.ds(base, _BPW)], f_v, sem)
    c4 = pltpu.async_copy(second_hbm.at[pl.ds(base, _BPW)], s_v, sem)
    c1.wait()
    c2.wait()
    c3.wait()
    c4.wait()

    for i in range(_BPW // _L):
        off = i * _L
        a = plsc.load_gather(t1_v, [f_v[pl.ds(off, _L)]])
        c = plsc.load_gather(t2_v, [s_v[pl.ds(off, _L)]])
        x = a + c
        o_v[pl.ds(off, _L)] = 1.0 / (1.0 + jnp.exp(-x))

    pltpu.sync_copy(o_v, out_hbm.at[pl.ds(base, _BPW)])


_sc_gather = functools.partial(
    pl.kernel,
    out_type=jax.ShapeDtypeStruct((_BATCH,), jnp.float32),
    mesh=plsc.VectorSubcoreMesh(core_axis_name="c", subcore_axis_name="s"),
    compiler_params=pltpu.CompilerParams(needs_layout_passes=False),
    scratch_types=[
        pltpu.VMEM((_VOCAB,), jnp.float32),
        pltpu.VMEM((_VOCAB,), jnp.float32),
        pltpu.VMEM((_BPW,), jnp.int32),
        pltpu.VMEM((_BPW,), jnp.int32),
        pltpu.VMEM((_BPW,), jnp.float32),
        pltpu.SemaphoreType.DMA,
    ],
)(_sc_gather_body)


@jax.jit
def kernel(first, second, table, W, b):
    w2row = W.reshape(2, _EMB)
    t1, t2 = _tc_scores(table, w2row, b)
    out = _sc_gather(t1, t2, first.astype(jnp.int32), second.astype(jnp.int32))
    return out.reshape(_BATCH, 1)


# speedup vs baseline: 5.2725x; 1.0389x over previous
import functools

import jax
import jax.numpy as jnp
from jax import lax
from jax.experimental import pallas as pl
from jax.experimental.pallas import tpu as pltpu
from jax.experimental.pallas import tpu_sc as plsc

_VOCAB = 800
_EMB = 128
_BATCH = 16384

_NC = 2
_NS = 16
_NW = _NC * _NS
_L = 16
_BPW = _BATCH // _NW


def _tc_scores_body(table_ref, w_ref, b_ref, t1_ref, t2_ref):
    tab = table_ref[...]
    w1 = w_ref[0:1, :]
    w2 = w_ref[1:2, :]
    t1_ref[...] = jnp.sum(tab * w1, axis=1) + b_ref[0]
    t2_ref[...] = jnp.sum(tab * w2, axis=1)


def _tc_scores(table, w2row, b):
    return pl.pallas_call(
        _tc_scores_body,
        out_shape=[
            jax.ShapeDtypeStruct((_VOCAB,), jnp.float32),
            jax.ShapeDtypeStruct((_VOCAB,), jnp.float32),
        ],
        in_specs=[
            pl.BlockSpec(memory_space=pltpu.VMEM),
            pl.BlockSpec(memory_space=pltpu.VMEM),
            pl.BlockSpec(memory_space=pltpu.SMEM),
        ],
    )(table, w2row, b)


def _sc_gather_body(t1_hbm, t2_hbm, first_hbm, second_hbm, out_hbm,
                    t1_v, t2_v, f_v, s_v, o_v, sem):
    wid = lax.axis_index("s") * _NC + lax.axis_index("c")
    base = wid * _BPW
    c1 = pltpu.async_copy(t1_hbm, t1_v, sem)
    c2 = pltpu.async_copy(t2_hbm, t2_v, sem)
    c3 = pltpu.async_copy(first_hbm.at[pl.ds(base, _BPW)], f_v, sem)
    c4 = pltpu.async_copy(second_hbm.at[pl.ds(base, _BPW)], s_v, sem)
    c1.wait()
    c2.wait()
    c3.wait()
    c4.wait()

    @plsc.parallel_loop(0, _BPW, _L, unroll=4)
    def _gather_step(off):
        a = plsc.load_gather(t1_v, [f_v[pl.ds(off, _L)]])
        c = plsc.load_gather(t2_v, [s_v[pl.ds(off, _L)]])
        x = a + c
        o_v[pl.ds(off, _L)] = 1.0 / (1.0 + jnp.exp(-x))

    pltpu.sync_copy(o_v, out_hbm.at[pl.ds(base, _BPW)])


_sc_gather = functools.partial(
    pl.kernel,
    out_type=jax.ShapeDtypeStruct((_BATCH,), jnp.float32),
    mesh=plsc.VectorSubcoreMesh(core_axis_name="c", subcore_axis_name="s"),
    compiler_params=pltpu.CompilerParams(needs_layout_passes=False),
    scratch_types=[
        pltpu.VMEM((_VOCAB,), jnp.float32),
        pltpu.VMEM((_VOCAB,), jnp.float32),
        pltpu.VMEM((_BPW,), jnp.int32),
        pltpu.VMEM((_BPW,), jnp.int32),
        pltpu.VMEM((_BPW,), jnp.float32),
        pltpu.SemaphoreType.DMA,
    ],
)(_sc_gather_body)


@jax.jit
def kernel(first, second, table, W, b):
    w2row = W.reshape(2, _EMB)
    t1, t2 = _tc_scores(table, w2row, b)
    out = _sc_gather(t1, t2, first.astype(jnp.int32), second.astype(jnp.int32))
    return out.reshape(_BATCH, 1)
